# baseline (device time: 227285 ns/iter reference)
import jax
import jax.numpy as jnp
from jax import lax
from jax.experimental import pallas as pl
from jax.experimental.pallas import tpu as pltpu

NZ = 4
CE = 192


def kernel(x, assign, W1, W2):
    T, D = x.shape
    EL, _, F = W1.shape
    E = NZ * EL
    B = EL * CE

    xb = x.astype(jnp.bfloat16)
    w1b = W1.astype(jnp.bfloat16)
    w2b = W2.astype(jnp.bfloat16)

    ar = assign.astype(jnp.int32)
    oh = (ar[:, None] == jnp.arange(E, dtype=jnp.int32)[None, :])
    ohi = oh.astype(jnp.int32)
    pos = jnp.sum(ohi * (jnp.cumsum(ohi, axis=0) - ohi), axis=1)
    slot = ar * CE + pos
    slot_r = slot.reshape(1, T)
    slot_c = slot.reshape(T, 1)

    def body(x_ref, sr_ref, sc_ref, w1_ref, w2_ref, out_ref,
             sbuf, rxb, rcol, w1e, w2e,
             ld, w1s, w2s, s1s, s1r, s2s, s2r):
        p = lax.axis_index("z")
        mx = lax.axis_index("x")
        my = lax.axis_index("y")

        xv = x_ref[...]
        srv = sr_ref[...]

        def dispatch(k):
            c = lax.rem(p + k, NZ)
            ii = lax.broadcasted_iota(jnp.int32, (B, T), 0) + c * B
            pt = (ii == srv).astype(jnp.bfloat16)
            sb = jnp.dot(pt, xv, preferred_element_type=jnp.float32)
            sbuf[k] = sb.astype(jnp.bfloat16)

        dispatch(2)

        barrier = pltpu.get_barrier_semaphore()
        for k in range(1, NZ):
            pl.semaphore_signal(
                barrier, inc=1,
                device_id=(mx, my, lax.rem(p + k, NZ)),
                device_id_type=pl.DeviceIdType.MESH)
        pl.semaphore_wait(barrier, NZ - 1)

        sends1 = {}
        for k in (2, 1, 3):
            if k != 2:
                dispatch(k)
            r = pltpu.make_async_remote_copy(
                src_ref=sbuf.at[k], dst_ref=rxb.at[NZ - k],
                send_sem=s1s.at[k], recv_sem=s1r.at[NZ - k],
                device_id=(mx, my, lax.rem(p + k, NZ)),
                device_id_type=pl.DeviceIdType.MESH)
            r.start()
            sends1[k] = r
        dispatch(0)
        cp = pltpu.make_async_copy(sbuf.at[0], rxb.at[0], ld.at[0])
        cp.start()

        def wload(e, buf):
            l1 = pltpu.make_async_copy(w1_ref.at[e], w1e.at[buf],
                                       w1s.at[buf])
            l1.start()
            l2 = pltpu.make_async_copy(w2_ref.at[e], w2e.at[buf],
                                       w2s.at[buf])
            l2.start()
            return l1, l2

        wl = wload(0, 0)

        cp.wait()
        for j in (1, 3, 2):
            pltpu.make_async_remote_copy(
                src_ref=sbuf.at[0], dst_ref=rxb.at[j],
                send_sem=s1s.at[j], recv_sem=s1r.at[j],
                device_id=(mx, my, p),
                device_id_type=pl.DeviceIdType.MESH).wait_recv()
        for k in (2, 1, 3):
            sends1[k].wait_send()

        for e in range(EL):
            buf = e % 2
            wl[0].wait()
            wl[1].wait()
            if e + 1 < EL:
                nxt = wload(e + 1, 1 - buf)
            xin = rxb[:, e * CE:(e + 1) * CE, :].reshape(NZ * CE, D)
            h = jnp.dot(xin, w1e[buf],
                        preferred_element_type=jnp.float32)
            hb = jnp.maximum(h, 0.0).astype(jnp.bfloat16)
            o = jnp.dot(hb, w2e[buf],
                        preferred_element_type=jnp.float32)
            sbuf[:, e * CE:(e + 1) * CE, :] = (
                o.astype(jnp.bfloat16).reshape(NZ, CE, D))
            if e + 1 < EL:
                wl = nxt

        sends2 = []
        for j in (2, 1, 3):
            r = pltpu.make_async_remote_copy(
                src_ref=sbuf.at[j], dst_ref=rcol.at[NZ - j],
                send_sem=s2s.at[j], recv_sem=s2r.at[NZ - j],
                device_id=(mx, my, lax.rem(p + j, NZ)),
                device_id_type=pl.DeviceIdType.MESH)
            r.start()
            sends2.append(r)

        scv = sc_ref[...]

        def combine(m, rows):
            c = lax.rem(p + m, NZ)
            jj = lax.broadcasted_iota(jnp.int32, (T, B), 1) + c * B
            pm = (scv == jj).astype(jnp.bfloat16)
            return jnp.dot(pm, rows, preferred_element_type=jnp.float32)

        out_ref[...] = combine(0, sbuf[0])
        for m in (1, 3, 2):
            pltpu.make_async_remote_copy(
                src_ref=sbuf.at[0], dst_ref=rcol.at[m],
                send_sem=s2s.at[m], recv_sem=s2r.at[m],
                device_id=(mx, my, p),
                device_id_type=pl.DeviceIdType.MESH).wait_recv()
            out_ref[...] = out_ref[...] + combine(m, rcol[m])

        for r in sends2:
            r.wait_send()

    cparams = pltpu.CompilerParams(
        collective_id=0, vmem_limit_bytes=100 * 1024 * 1024)

    return pl.pallas_call(
        body,
        out_shape=jax.ShapeDtypeStruct((T, D), jnp.float32),
        in_specs=[
            pl.BlockSpec(memory_space=pltpu.VMEM),
            pl.BlockSpec(memory_space=pltpu.VMEM),
            pl.BlockSpec(memory_space=pltpu.VMEM),
            pl.BlockSpec(memory_space=pl.ANY),
            pl.BlockSpec(memory_space=pl.ANY),
        ],
        out_specs=pl.BlockSpec(memory_space=pltpu.VMEM),
        scratch_shapes=[
            pltpu.VMEM((NZ, B, D), jnp.bfloat16),
            pltpu.VMEM((NZ, B, D), jnp.bfloat16),
            pltpu.VMEM((NZ, B, D), jnp.bfloat16),
            pltpu.VMEM((2, D, F), jnp.bfloat16),
            pltpu.VMEM((2, F, D), jnp.bfloat16),
            pltpu.SemaphoreType.DMA((1,)),
            pltpu.SemaphoreType.DMA((2,)),
            pltpu.SemaphoreType.DMA((2,)),
            pltpu.SemaphoreType.DMA((NZ,)),
            pltpu.SemaphoreType.DMA((NZ,)),
            pltpu.SemaphoreType.DMA((NZ,)),
            pltpu.SemaphoreType.DMA((NZ,)),
        ],
        compiler_params=cparams,
    )(xb, slot_r, slot_c, w1b, w2b)
